# baseline (speedup 1.0000x reference)
import functools

import jax
import jax.numpy as jnp
from jax.experimental import pallas as pl
from jax.experimental.pallas import tpu as pltpu

N, B, NV, C = 100000, 64, 2, 128
T = 20000
H = 5
TH = T // H
NB = N // T


def _fused_kernel(nf0_ref, nf1_ref, nf2_ref, nf3_ref, nf4_ref, batch_ref, mask_ref, vn_ref, w1_ref, b1_ref,
                  w2_ref, b2_ref, w3_ref, b3_ref, out_ref,
                  cnt_ref, vnlin_ref):
    step = pl.program_id(0)

    @pl.when(step == 0)
    def _init():
        w1_bot = w1_ref[C:, :]
        b1 = b1_ref[...]
        for v in range(NV):
            vnlin_ref[:, v * C:(v + 1) * C] = (
                jnp.dot(vn_ref[:, v * C:(v + 1) * C], w1_bot,
                        preferred_element_type=jnp.float32) + b1)
        out_ref[...] = jnp.zeros_like(out_ref)
        cnt_ref[...] = jnp.zeros_like(cnt_ref)

    w1_top = w1_ref[:C, :]
    w2 = w2_ref[...]
    b2 = b2_ref[...]
    vnlin = vnlin_ref[...]
    iota_b = jax.lax.broadcasted_iota(jnp.int32, (B, TH), 0)

    scat = [[None] * H for _ in range(NV)]
    cnt_parts = [None] * H
    nf_refs = (nf0_ref, nf1_ref, nf2_ref, nf3_ref, nf4_ref)
    for t in range(H):
        nf = nf_refs[t][...]
        batch_row = batch_ref[0, :, pl.ds(t * TH, TH)]
        mask_row = mask_ref[0, :, pl.ds(t * TH, TH)]

        oh = (iota_b == batch_row).astype(jnp.float32)
        ohm = oh * mask_row

        a = jnp.dot(nf, w1_top, preferred_element_type=jnp.float32)
        g = jax.lax.dot_general(oh, vnlin,
                                (((0,), (0,)), ((), ())),
                                preferred_element_type=jnp.float32)

        for v in range(NV):
            h1 = jnp.maximum(a + g[:, v * C:(v + 1) * C], 0.0)
            h2 = jnp.maximum(
                jnp.dot(h1, w2, preferred_element_type=jnp.float32) + b2,
                0.0)
            scat[v][t] = jnp.dot(ohm, h2,
                                 preferred_element_type=jnp.float32)
        cnt_parts[t] = jnp.sum(ohm, axis=1, keepdims=True)

    for v in range(NV):
        out_ref[:, v * C:(v + 1) * C] += sum(scat[v][1:], scat[v][0])
    cnt_ref[...] += jnp.broadcast_to(
        sum(cnt_parts[1:], cnt_parts[0]), (B, C))

    @pl.when(step == NB - 1)
    def _finalize():
        cnt = cnt_ref[...]
        denom = jnp.maximum(cnt, 1.0)
        w3 = w3_ref[...]
        b3 = b3_ref[...]
        acc = out_ref[...]
        res = []
        for v in range(NV):
            s = jnp.dot(acc[:, v * C:(v + 1) * C], w3,
                        preferred_element_type=jnp.float32)
            res.append((s + b3 * cnt) / denom)
        out_ref[...] = jnp.concatenate(res, axis=1)


@functools.partial(jax.jit, static_argnames=("interpret",))
def _run(node_features, vn2, batch2, mask2, W1, b1, W2, b2, W3, b3,
         interpret=False):
    out = pl.pallas_call(
        _fused_kernel,
        grid=(NB,),
        in_specs=[
            pl.BlockSpec((TH, C), lambda i: (H * i + 0, 0)),
            pl.BlockSpec((TH, C), lambda i: (H * i + 1, 0)),
            pl.BlockSpec((TH, C), lambda i: (H * i + 2, 0)),
            pl.BlockSpec((TH, C), lambda i: (H * i + 3, 0)),
            pl.BlockSpec((TH, C), lambda i: (H * i + 4, 0)),
            pl.BlockSpec((1, 1, T), lambda i: (i, 0, 0)),
            pl.BlockSpec((1, 1, T), lambda i: (i, 0, 0)),
            pl.BlockSpec((B, NV * C), lambda i: (0, 0)),
            pl.BlockSpec((2 * C, C), lambda i: (0, 0)),
            pl.BlockSpec((1, C), lambda i: (0, 0)),
            pl.BlockSpec((C, C), lambda i: (0, 0)),
            pl.BlockSpec((1, C), lambda i: (0, 0)),
            pl.BlockSpec((C, C), lambda i: (0, 0)),
            pl.BlockSpec((1, C), lambda i: (0, 0)),
        ],
        out_specs=pl.BlockSpec((B, NV * C), lambda i: (0, 0)),
        out_shape=jax.ShapeDtypeStruct((B, NV * C), jnp.float32),
        scratch_shapes=[
            pltpu.VMEM((B, C), jnp.float32),
            pltpu.VMEM((B, NV * C), jnp.float32),
        ],
        compiler_params=pltpu.CompilerParams(
            dimension_semantics=("arbitrary",),
        ),
        interpret=interpret,
    )(node_features, node_features, node_features, node_features, node_features,
      batch2, mask2, vn2, W1, b1, W2, b2, W3, b3)
    return out.reshape(B, NV, C)


def kernel(node_features, vn_features, batch, node_mask, W1, b1, W2, b2,
           W3, b3):
    vn2 = vn_features.reshape(B, NV * C)
    batch2 = batch.astype(jnp.int32).reshape(NB, 1, T)
    mask2 = node_mask.astype(jnp.float32).reshape(NB, 1, T)
    return _run(node_features, vn2, batch2, mask2,
                W1, b1.reshape(1, C), W2, b2.reshape(1, C),
                W3, b3.reshape(1, C))


# speedup vs baseline: 1.0194x; 1.0194x over previous
import functools

import jax
import jax.numpy as jnp
from jax.experimental import pallas as pl
from jax.experimental.pallas import tpu as pltpu

N, B, NV, C = 100000, 64, 2, 128
T = 20000
H = 5
TH = T // H
NB = N // T


def _fused_kernel(nf_ref, batch_ref, mask_ref, vn_ref, w1_ref, b1_ref,
                  w2_ref, b2_ref, w3_ref, b3_ref, out_ref,
                  cnt_ref, vnlin_ref):
    step = pl.program_id(0)

    @pl.when(step == 0)
    def _init():
        w1_bot = w1_ref[C:, :]
        b1 = b1_ref[...]
        for v in range(NV):
            vnlin_ref[:, v * C:(v + 1) * C] = (
                jnp.dot(vn_ref[:, v * C:(v + 1) * C], w1_bot,
                        preferred_element_type=jnp.float32) + b1)
        out_ref[...] = jnp.zeros_like(out_ref)
        cnt_ref[...] = jnp.zeros_like(cnt_ref)

    w1_top = w1_ref[:C, :]
    w2 = w2_ref[...]
    b2 = b2_ref[...]
    vnlin = vnlin_ref[...]
    iota_b = jax.lax.broadcasted_iota(jnp.int32, (B, TH), 0)

    scat = [[None] * H for _ in range(NV)]
    cnt_parts = [None] * H
    for t in range(H):
        nf = nf_ref[pl.ds(t * TH, TH), :]
        batch_row = batch_ref[0, :, pl.ds(t * TH, TH)]
        mask_row = mask_ref[0, :, pl.ds(t * TH, TH)]

        oh = (iota_b == batch_row).astype(jnp.float32)
        ohm = oh * mask_row

        a = jnp.dot(nf, w1_top, preferred_element_type=jnp.float32)
        g = jax.lax.dot_general(oh, vnlin,
                                (((0,), (0,)), ((), ())),
                                preferred_element_type=jnp.float32)

        for v in range(NV):
            h1 = jnp.maximum(a + g[:, v * C:(v + 1) * C], 0.0)
            h2 = jnp.maximum(
                jnp.dot(h1, w2, preferred_element_type=jnp.float32) + b2,
                0.0)
            scat[v][t] = jnp.dot(ohm, h2,
                                 preferred_element_type=jnp.float32)
        cnt_parts[t] = jnp.sum(ohm, axis=1, keepdims=True)

    for v in range(NV):
        out_ref[:, v * C:(v + 1) * C] += sum(scat[v][1:], scat[v][0])
    cnt_ref[...] += jnp.broadcast_to(
        sum(cnt_parts[1:], cnt_parts[0]), (B, C))

    @pl.when(step == NB - 1)
    def _finalize():
        cnt = cnt_ref[...]
        denom = jnp.maximum(cnt, 1.0)
        w3 = w3_ref[...]
        b3 = b3_ref[...]
        acc = out_ref[...]
        res = []
        for v in range(NV):
            s = jnp.dot(acc[:, v * C:(v + 1) * C], w3,
                        preferred_element_type=jnp.float32)
            res.append((s + b3 * cnt) / denom)
        out_ref[...] = jnp.concatenate(res, axis=1)


@functools.partial(jax.jit, static_argnames=("interpret",))
def _run(node_features, vn2, batch2, mask2, W1, b1, W2, b2, W3, b3,
         interpret=False):
    out = pl.pallas_call(
        _fused_kernel,
        grid=(NB,),
        in_specs=[
            pl.BlockSpec((T, C), lambda i: (i, 0)),
            pl.BlockSpec((1, 1, T), lambda i: (i, 0, 0)),
            pl.BlockSpec((1, 1, T), lambda i: (i, 0, 0)),
            pl.BlockSpec((B, NV * C), lambda i: (0, 0)),
            pl.BlockSpec((2 * C, C), lambda i: (0, 0)),
            pl.BlockSpec((1, C), lambda i: (0, 0)),
            pl.BlockSpec((C, C), lambda i: (0, 0)),
            pl.BlockSpec((1, C), lambda i: (0, 0)),
            pl.BlockSpec((C, C), lambda i: (0, 0)),
            pl.BlockSpec((1, C), lambda i: (0, 0)),
        ],
        out_specs=pl.BlockSpec((B, NV * C), lambda i: (0, 0)),
        out_shape=jax.ShapeDtypeStruct((B, NV * C), jnp.float32),
        scratch_shapes=[
            pltpu.VMEM((B, C), jnp.float32),
            pltpu.VMEM((B, NV * C), jnp.float32),
        ],
        compiler_params=pltpu.CompilerParams(
            dimension_semantics=("arbitrary",),
        ),
        interpret=interpret,
    )(node_features, batch2, mask2, vn2, W1, b1, W2, b2, W3, b3)
    return out.reshape(B, NV, C)


def kernel(node_features, vn_features, batch, node_mask, W1, b1, W2, b2,
           W3, b3):
    vn2 = vn_features.reshape(B, NV * C)
    batch2 = batch.astype(jnp.int32).reshape(NB, 1, T)
    mask2 = node_mask.astype(jnp.float32).reshape(NB, 1, T)
    return _run(node_features, vn2, batch2, mask2,
                W1, b1.reshape(1, C), W2, b2.reshape(1, C),
                W3, b3.reshape(1, C))
